# Initial kernel scaffold; baseline (speedup 1.0000x reference)
import jax
import jax.numpy as jnp
from jax.experimental import pallas as pl


def kernel(gt_bboxes, gt_classes, bbox_preds, cls_preds, obj_preds, expanded_strides, x_shifts, y_shifts):
    raise NotImplementedError("write your pallas kernel here")


# speedup vs baseline: 9.9935x; 9.9935x over previous
import functools

import jax
import jax.numpy as jnp
from jax import lax
from jax.experimental import pallas as pl
from jax.experimental.pallas import tpu as pltpu
from jax.experimental.pallas import tpu_sc as plsc

A_REAL = 20000
A_PAD = 20480
BLK = 2048
N_BLK = A_PAD // BLK
G_PAD = 128
N_CAND = 10
DUMMY = A_PAD - 1
BIG = 1e37
BIGI = 1 << 30
N_SC = 32
CHUNK = A_PAD // N_SC
P_PAD = 12 * 128


def _extract_topk_cost(w, iou_mat, row_iota, blk_off):
    vals, idxs, ious = [], [], []
    for _ in range(N_CAND):
        m = jnp.min(w, axis=0, keepdims=True)
        r = jnp.min(jnp.where(w == m, row_iota, BIGI), axis=0, keepdims=True)
        sel = row_iota == r
        vals.append(m)
        idxs.append(r + blk_off)
        ious.append(jnp.sum(jnp.where(sel, iou_mat, 0.0), axis=0, keepdims=True))
        w = jnp.where(sel, BIG, w)
    return (jnp.concatenate(vals, 0), jnp.concatenate(idxs, 0),
            jnp.concatenate(ious, 0))


def _merge_topk_cost(cv, ci, cu):
    n = cv.shape[0]
    row = lax.broadcasted_iota(jnp.int32, (n, G_PAD), 0)
    vals, idxs, ious = [], [], []
    for _ in range(N_CAND):
        m = jnp.min(cv, axis=0, keepdims=True)
        eq = cv == m
        imin = jnp.min(jnp.where(eq, ci, BIGI), axis=0, keepdims=True)
        sel = eq & (ci == imin)
        vals.append(m)
        idxs.append(imin)
        ious.append(jnp.sum(jnp.where(sel, cu, 0.0), axis=0, keepdims=True))
        cv = jnp.where(sel, BIG, cv)
    return (jnp.concatenate(vals, 0), jnp.concatenate(idxs, 0),
            jnp.concatenate(ious, 0))


def _extract_topk_max(w, row_iota):
    vals = []
    for _ in range(N_CAND):
        m = jnp.max(w, axis=0, keepdims=True)
        r = jnp.min(jnp.where(w == m, row_iota, BIGI), axis=0, keepdims=True)
        vals.append(m)
        w = jnp.where(row_iota == r, -BIG, w)
    return jnp.concatenate(vals, 0)


def _phase1_body(gt_ref, gtc_ref, pk_ref, cls_ref,
                 bg_ref, pidx_ref, pval_ref,
                 rc_ref, ri_ref, ru_ref, rt_ref):
    i = pl.program_id(0)
    blk_off = i * BLK

    @pl.when(i == 0)
    def _init():
        rc_ref[...] = jnp.full((N_CAND, G_PAD), BIG, jnp.float32)
        ri_ref[...] = jnp.full((N_CAND, G_PAD), DUMMY, jnp.int32)
        ru_ref[...] = jnp.zeros((N_CAND, G_PAD), jnp.float32)
        rt_ref[...] = jnp.zeros((N_CAND, G_PAD), jnp.float32)

    pk = pk_ref[...]
    cx, cy = pk[:, 0:1], pk[:, 1:2]
    w_, h_ = pk[:, 2:3], pk[:, 3:4]
    xs, ys = pk[:, 4:5], pk[:, 5:6]
    st, obj = pk[:, 6:7], pk[:, 7:8]

    gx, gy = gt_ref[0:1, :], gt_ref[1:2, :]
    gw, gh = gt_ref[2:3, :], gt_ref[3:4, :]

    lane_g = lax.broadcasted_iota(jnp.int32, (1, G_PAD), 1)
    valid_g = lane_g < 100
    row_iota = lax.broadcasted_iota(jnp.int32, (BLK, G_PAD), 0)
    arow = lax.broadcasted_iota(jnp.int32, (BLK, 1), 0) + blk_off
    valid_a = arow < A_REAL

    xcen = xs * st + 0.5 * st
    ycen = ys * st + 0.5 * st
    gl, gr = gx - 0.5 * gw, gx + 0.5 * gw
    gtt, gbb = gy - 0.5 * gh, gy + 0.5 * gh
    in_boxes = jnp.minimum(jnp.minimum(jnp.minimum(xcen - gl, ycen - gtt),
                                       gr - xcen), gbb - ycen) > 0.0
    cl, cr = gx - 2.5 * st, gx + 2.5 * st
    ct, cb = gy - 2.5 * st, gy + 2.5 * st
    in_centers = jnp.minimum(jnp.minimum(jnp.minimum(xcen - cl, ycen - ct),
                                         cr - xcen), cb - ycen) > 0.0
    ibc = in_boxes & in_centers
    any_f = jnp.max(jnp.where(valid_g & (in_boxes | in_centers), 1.0, 0.0),
                    axis=1, keepdims=True)
    fg = (any_f > 0.5) & valid_a

    tlx = jnp.maximum(gl, cx - 0.5 * w_)
    tly = jnp.maximum(gtt, cy - 0.5 * h_)
    brx = jnp.minimum(gr, cx + 0.5 * w_)
    bry = jnp.minimum(gbb, cy + 0.5 * h_)
    en = ((tlx < brx) & (tly < bry)).astype(jnp.float32)
    area_i = (brx - tlx) * (bry - tly) * en
    area_g = gw * gh
    area_p = w_ * h_
    iou = area_i / (area_g + area_p - area_i + 1e-12)
    iou = jnp.where(fg, iou, 0.0)
    iou_loss = -jnp.log(iou + 1e-08)

    cls = cls_ref[...]
    p = jnp.sqrt(jax.nn.sigmoid(cls) * jax.nn.sigmoid(obj))
    log1m = jnp.maximum(jnp.log(1.0 - p), -100.0)
    logp = jnp.maximum(jnp.log(p), -100.0)
    c_iota = lax.broadcasted_iota(jnp.int32, (cls.shape[1], G_PAD), 0)
    w_onehot = (c_iota == gtc_ref[...]).astype(jnp.float32)
    pos = lax.dot_general(log1m - logp, w_onehot,
                          (((1,), (0,)), ((), ())),
                          precision=lax.Precision.HIGHEST,
                          preferred_element_type=jnp.float32)
    cls_loss = pos - jnp.sum(log1m, axis=1, keepdims=True)

    cost = (cls_loss + 3.0 * iou_loss
            + 100000.0 * jnp.where(ibc, 0.0, 1.0)
            + 1000000.0 * jnp.where(fg, 0.0, 1.0))
    cost = jnp.where(valid_a, cost, BIG)

    cost_gm = jnp.where(valid_g, cost, BIG)
    cmin = jnp.min(cost_gm, axis=1, keepdims=True)
    amin_lane = jnp.min(jnp.where(cost_gm == cmin, lane_g, BIGI),
                        axis=1, keepdims=True)
    bg_ref[...] = jnp.sum(jnp.where(lane_g == amin_lane, iou, 0.0),
                          axis=1, keepdims=True)

    bv, bi, bu = _extract_topk_cost(cost, iou, row_iota, blk_off)
    nv, ni, nu = _merge_topk_cost(jnp.concatenate([rc_ref[...], bv], 0),
                                  jnp.concatenate([ri_ref[...], bi], 0),
                                  jnp.concatenate([ru_ref[...], bu], 0))
    rc_ref[...], ri_ref[...], ru_ref[...] = nv, ni, nu

    bt = _extract_topk_max(iou, row_iota)
    nt = _extract_topk_max(jnp.concatenate([rt_ref[...], bt], 0),
                           lax.broadcasted_iota(jnp.int32, (2 * N_CAND, G_PAD), 0))
    rt_ref[...] = nt

    dk = jnp.maximum(jnp.sum(nt, axis=0, keepdims=True).astype(jnp.int32), 1)
    slot = lax.broadcasted_iota(jnp.int32, (N_CAND, G_PAD), 0)
    lane2 = lax.broadcasted_iota(jnp.int32, (N_CAND, G_PAD), 1)
    keep = (slot < dk) & (lane2 < 100)
    pidx_ref[...] = jnp.concatenate(
        [jnp.where(keep, ni, DUMMY),
         jnp.full((2, G_PAD), DUMMY, jnp.int32)], 0)
    pval_ref[...] = jnp.concatenate(
        [jnp.where(keep, nu, 0.0), jnp.zeros((2, G_PAD), jnp.float32)], 0)


def _phase2_sc(idx_hbm, val_hbm, bg_hbm, out_hbm,
               idx_v, val_v, bg_v, cnt_v, sum_v, out_v):
    wid = lax.axis_index("s") * 2 + lax.axis_index("c")
    base = wid * CHUNK
    pltpu.sync_copy(idx_hbm, idx_v)
    pltpu.sync_copy(val_hbm, val_v)
    pltpu.sync_copy(bg_hbm.at[pl.ds(base, CHUNK)], bg_v)
    zeros16 = jnp.zeros((16,), jnp.float32)
    for k in range(CHUNK // 16):
        cnt_v[pl.ds(k * 16, 16)] = zeros16
        sum_v[pl.ds(k * 16, 16)] = zeros16
    ones16 = jnp.ones((16,), jnp.float32)
    for q in range(P_PAD // 16):
        gi = idx_v[pl.ds(q * 16, 16)]
        local = gi - base
        m = (gi >= base) & (gi < base + CHUNK)
        v = val_v[pl.ds(q * 16, 16)]
        plsc.addupdate_scatter(cnt_v, [local], ones16, mask=m)
        plsc.addupdate_scatter(sum_v, [local], v, mask=m)
    for k in range(CHUNK // 16):
        s = pl.ds(k * 16, 16)
        out_v[s] = jnp.where(cnt_v[s] > 1.5, bg_v[s], sum_v[s])
    pltpu.sync_copy(out_v, out_hbm.at[pl.ds(base, CHUNK)])


@jax.jit
def kernel(gt_bboxes, gt_classes, bbox_preds, cls_preds, obj_preds,
           expanded_strides, x_shifts, y_shifts):
    f32 = jnp.float32
    packed = jnp.concatenate(
        [bbox_preds[0].astype(f32),
         x_shifts[0][:, None].astype(f32),
         y_shifts[0][:, None].astype(f32),
         expanded_strides[0][:, None].astype(f32),
         obj_preds[0].astype(f32)], axis=1)
    packed = jnp.pad(packed, ((0, A_PAD - A_REAL), (0, 0)))
    cls_p = jnp.pad(cls_preds[0].astype(f32), ((0, A_PAD - A_REAL), (0, 0)))
    gt_t = jnp.pad(gt_bboxes.T.astype(f32), ((0, 0), (0, G_PAD - 100)))
    gtc = jnp.pad(gt_classes.astype(jnp.int32), (0, G_PAD - 100))[None, :]

    bg, pidx, pval = pl.pallas_call(
        _phase1_body,
        grid=(N_BLK,),
        in_specs=[
            pl.BlockSpec((4, G_PAD), lambda i: (0, 0)),
            pl.BlockSpec((1, G_PAD), lambda i: (0, 0)),
            pl.BlockSpec((BLK, 8), lambda i: (i, 0)),
            pl.BlockSpec((BLK, cls_p.shape[1]), lambda i: (i, 0)),
        ],
        out_specs=[
            pl.BlockSpec((BLK, 1), lambda i: (i, 0)),
            pl.BlockSpec((12, G_PAD), lambda i: (0, 0)),
            pl.BlockSpec((12, G_PAD), lambda i: (0, 0)),
        ],
        out_shape=[
            jax.ShapeDtypeStruct((A_PAD, 1), f32),
            jax.ShapeDtypeStruct((12, G_PAD), jnp.int32),
            jax.ShapeDtypeStruct((12, G_PAD), f32),
        ],
        scratch_shapes=[
            pltpu.VMEM((N_CAND, G_PAD), f32),
            pltpu.VMEM((N_CAND, G_PAD), jnp.int32),
            pltpu.VMEM((N_CAND, G_PAD), f32),
            pltpu.VMEM((N_CAND, G_PAD), f32),
        ],
    )(gt_t, gtc, packed, cls_p)

    sc_call = pl.kernel(
        _phase2_sc,
        mesh=plsc.VectorSubcoreMesh(core_axis_name="c", subcore_axis_name="s"),
        compiler_params=pltpu.CompilerParams(needs_layout_passes=False),
        out_type=jax.ShapeDtypeStruct((A_PAD,), f32),
        scratch_types=[
            pltpu.VMEM((P_PAD,), jnp.int32),
            pltpu.VMEM((P_PAD,), f32),
            pltpu.VMEM((CHUNK,), f32),
            pltpu.VMEM((CHUNK,), f32),
            pltpu.VMEM((CHUNK,), f32),
            pltpu.VMEM((CHUNK,), f32),
        ],
    )
    out = sc_call(pidx.reshape(P_PAD), pval.reshape(P_PAD),
                  bg.reshape(A_PAD))
    return out[:A_REAL]
